# baseline (speedup 1.0000x reference)
import jax
import jax.numpy as jnp
import numpy as np
from jax.experimental import pallas as pl

_MININT = np.int32(-2147483648)
_MAXPOS = np.int32(2147483647)


def _dott(x, w):
    return jax.lax.dot_general(x, w, (((1,), (1,)), ((), ())),
                               preferred_element_type=jnp.float32)


def _kwta(x, key_src, k, tri_bf16):
    skey = jax.lax.bitcast_convert_type(key_src + 0.0, jnp.int32)
    skey = jnp.where(skey < 0, skey ^ _MAXPOS, skey)

    def body(i, t_u):
        bit = jax.lax.shift_left(jnp.int32(1), jnp.int32(31) - i)
        cand = t_u | bit
        thr = cand ^ _MININT
        cnt = jnp.sum((skey >= thr).astype(jnp.int32), axis=1, keepdims=True)
        return jnp.where(cnt >= k, cand, t_u)

    t_u = jax.lax.fori_loop(0, 32, body, jnp.zeros_like(k))
    t_s = t_u ^ _MININT

    gt = skey > t_s
    c_gt = jnp.sum(gt.astype(jnp.int32), axis=1, keepdims=True)
    eq = skey == t_s
    cum_excl = jnp.dot(eq.astype(jnp.bfloat16), tri_bf16,
                       preferred_element_type=jnp.float32)
    keep = eq & (cum_excl < (k - c_gt).astype(jnp.float32))
    mask = (gt | keep) & (k > 0)
    return jnp.where(mask, x, x / 3.0)


def _body(a_ref, w11_ref, b11_ref, w12_ref, b12_ref,
          w21_ref, b21_ref, w22_ref, b22_ref,
          w31_ref, b31_ref, w32_ref, b32_ref,
          wl1_ref, bl1_ref, wl2_ref, bl2_ref,
          wl3_ref, bl3_ref, wl4_ref, bl4_ref,
          t1_ref, t2_ref, t3_ref, out_ref):
    f32 = jnp.float32
    a = a_ref[...]

    h1 = jnp.tanh(_dott(a, w11_ref[...]) + b11_ref[...])
    z1 = _dott(h1.astype(jnp.bfloat16), w12_ref[...]) + b12_ref[...]
    k1 = jnp.argmax(z1, axis=1).astype(jnp.int32)[:, None]

    h2 = jnp.tanh(_dott(a, w21_ref[...]) + b21_ref[...])
    z2 = _dott(h2.astype(jnp.bfloat16), w22_ref[...]) + b22_ref[...]
    k2 = jnp.argmax(z2, axis=1).astype(jnp.int32)[:, None]

    h3 = jnp.tanh(_dott(a, w31_ref[...]) + b31_ref[...])
    z3 = _dott(h3.astype(jnp.bfloat16), w32_ref[...]) + b32_ref[...]
    k3 = jnp.argmax(z3, axis=1).astype(jnp.int32)[:, None]

    x = _dott(a, wl1_ref[...]) + bl1_ref[...]
    x = _kwta(x, x, k1, t1_ref[...])
    x = _dott(x.astype(jnp.bfloat16), wl2_ref[...]) + bl2_ref[...]
    x = _kwta(x, x, k2, t2_ref[...])
    x = _dott(x.astype(jnp.bfloat16), wl3_ref[...]) + bl3_ref[...]
    col = jax.lax.broadcasted_iota(jnp.int32, x.shape, 1)
    key3 = jnp.where(col < 64, x, f32(-1e30))
    x = _kwta(x, key3, k3, t3_ref[...])
    out_ref[...] = _dott(x.astype(jnp.bfloat16), wl4_ref[...]) + bl4_ref[...]


def _tri(n):
    r = jnp.arange(n, dtype=jnp.int32)
    return (r[:, None] < r[None, :]).astype(jnp.bfloat16)


def _wk(W, kpad):
    return jnp.pad(W, ((0, 0), (0, kpad - W.shape[1]))).astype(jnp.bfloat16)


def kernel(state, task_indicator,
           W_cx1_1, b_cx1_1, W_cx1_2, b_cx1_2,
           W_cx2_1, b_cx2_1, W_cx2_2, b_cx2_2,
           W_cx3_1, b_cx3_1, W_cx3_2, b_cx3_2,
           W_lin1, b_lin1, W_lin2, b_lin2,
           W_lin3, b_lin3, W_lin4, b_lin4):
    B = state.shape[0]
    INP = state.shape[1] + task_indicator.shape[1]
    KPAD = ((INP + 127) // 128) * 128
    R = 256
    H2, H1, NH = 1024, 512, 64

    a = jnp.concatenate(
        [state, task_indicator,
         jnp.zeros((B, KPAD - INP), jnp.float32)], axis=1
    ).astype(jnp.bfloat16)

    w11 = _wk(W_cx1_1, KPAD)
    b11 = b_cx1_1[None, :]
    w21 = _wk(W_cx2_1, KPAD)
    b21 = b_cx2_1[None, :]
    w31 = jnp.pad(_wk(W_cx3_1, KPAD), ((0, 64), (0, 0)))
    b31 = jnp.pad(b_cx3_1, (0, 64))[None, :]
    wl1 = _wk(W_lin1, KPAD)
    bl1 = b_lin1[None, :]

    w12 = W_cx1_2.astype(jnp.bfloat16)
    b12 = b_cx1_2[None, :]
    w22 = W_cx2_2.astype(jnp.bfloat16)
    b22 = b_cx2_2[None, :]
    w32 = jnp.pad(W_cx3_2, ((0, 64), (0, 64))).astype(jnp.bfloat16)
    b32 = jnp.pad(b_cx3_2, (0, 64), constant_values=-1e9)[None, :]
    wl2 = W_lin2.astype(jnp.bfloat16)
    bl2 = b_lin2[None, :]
    wl3 = jnp.pad(W_lin3, ((0, 64), (0, 0))).astype(jnp.bfloat16)
    bl3 = jnp.pad(b_lin3, (0, 64))[None, :]
    wl4 = jnp.pad(W_lin4, ((0, 0), (0, 64))).astype(jnp.bfloat16)
    bl4 = b_lin4[None, :]

    t1, t2, t3 = _tri(H2), _tri(H1), _tri(128)

    def const(shape):
        return pl.BlockSpec(shape, lambda i: (0, 0))

    out = pl.pallas_call(
        _body,
        grid=(B // R,),
        in_specs=[
            pl.BlockSpec((R, KPAD), lambda i: (i, 0)),
            const(w11.shape), const(b11.shape), const(w12.shape), const(b12.shape),
            const(w21.shape), const(b21.shape), const(w22.shape), const(b22.shape),
            const(w31.shape), const(b31.shape), const(w32.shape), const(b32.shape),
            const(wl1.shape), const(bl1.shape), const(wl2.shape), const(bl2.shape),
            const(wl3.shape), const(bl3.shape), const(wl4.shape), const(bl4.shape),
            const(t1.shape), const(t2.shape), const(t3.shape),
        ],
        out_specs=pl.BlockSpec((R, NH), lambda i: (i, 0)),
        out_shape=jax.ShapeDtypeStruct((B, NH), jnp.float32),
    )(a, w11, b11, w12, b12, w21, b21, w22, b22, w31, b31, w32, b32,
      wl1, bl1, wl2, bl2, wl3, bl3, wl4, bl4, t1, t2, t3)
    return out


# speedup vs baseline: 1.2578x; 1.2578x over previous
import jax
import jax.numpy as jnp
import numpy as np
from jax.experimental import pallas as pl

_MININT = np.int32(-2147483648)
_MAXPOS = np.int32(2147483647)


def _dott(x, w):
    return jax.lax.dot_general(x, w, (((1,), (1,)), ((), ())),
                               preferred_element_type=jnp.float32)


def _kwta(x, key_src, k, tri_bf16):
    skey = jax.lax.bitcast_convert_type(key_src + 0.0, jnp.int32)
    skey = jnp.where(skey < 0, skey ^ _MAXPOS, skey)

    skey_t = skey.T
    k_row = k.T

    def body(i, t_u):
        bit = jax.lax.shift_left(jnp.int32(1), jnp.int32(31) - i)
        cand = t_u | bit
        thr = cand ^ _MININT
        cnt = jnp.sum((skey_t >= thr).astype(jnp.int32), axis=0, keepdims=True)
        return jnp.where(cnt >= k_row, cand, t_u)

    t_u = jax.lax.fori_loop(0, 32, body, jnp.zeros_like(k_row))
    t_s = (t_u ^ _MININT).T

    gt = skey > t_s
    c_gt = jnp.sum(gt.astype(jnp.int32), axis=1, keepdims=True)
    eq = skey == t_s
    cum_excl = jnp.dot(eq.astype(jnp.bfloat16), tri_bf16,
                       preferred_element_type=jnp.float32)
    keep = eq & (cum_excl < (k - c_gt).astype(jnp.float32))
    mask = (gt | keep) & (k > 0)
    return jnp.where(mask, x, x / 3.0)


def _body(a_ref, w11_ref, b11_ref, w12_ref, b12_ref,
          w21_ref, b21_ref, w22_ref, b22_ref,
          w31_ref, b31_ref, w32_ref, b32_ref,
          wl1_ref, bl1_ref, wl2_ref, bl2_ref,
          wl3_ref, bl3_ref, wl4_ref, bl4_ref,
          t1_ref, t2_ref, t3_ref, out_ref):
    f32 = jnp.float32
    a = a_ref[...]

    h1 = jnp.tanh(_dott(a, w11_ref[...]) + b11_ref[...])
    z1 = _dott(h1.astype(jnp.bfloat16), w12_ref[...]) + b12_ref[...]
    k1 = jnp.argmax(z1, axis=1).astype(jnp.int32)[:, None]

    h2 = jnp.tanh(_dott(a, w21_ref[...]) + b21_ref[...])
    z2 = _dott(h2.astype(jnp.bfloat16), w22_ref[...]) + b22_ref[...]
    k2 = jnp.argmax(z2, axis=1).astype(jnp.int32)[:, None]

    h3 = jnp.tanh(_dott(a, w31_ref[...]) + b31_ref[...])
    z3 = _dott(h3.astype(jnp.bfloat16), w32_ref[...]) + b32_ref[...]
    k3 = jnp.argmax(z3, axis=1).astype(jnp.int32)[:, None]

    x = _dott(a, wl1_ref[...]) + bl1_ref[...]
    x = _kwta(x, x, k1, t1_ref[...])
    x = _dott(x.astype(jnp.bfloat16), wl2_ref[...]) + bl2_ref[...]
    x = _kwta(x, x, k2, t2_ref[...])
    x = _dott(x.astype(jnp.bfloat16), wl3_ref[...]) + bl3_ref[...]
    col = jax.lax.broadcasted_iota(jnp.int32, x.shape, 1)
    key3 = jnp.where(col < 64, x, f32(-1e30))
    x = _kwta(x, key3, k3, t3_ref[...])
    out_ref[...] = _dott(x.astype(jnp.bfloat16), wl4_ref[...]) + bl4_ref[...]


def _tri(n):
    r = jnp.arange(n, dtype=jnp.int32)
    return (r[:, None] < r[None, :]).astype(jnp.bfloat16)


def _wk(W, kpad):
    return jnp.pad(W, ((0, 0), (0, kpad - W.shape[1]))).astype(jnp.bfloat16)


def kernel(state, task_indicator,
           W_cx1_1, b_cx1_1, W_cx1_2, b_cx1_2,
           W_cx2_1, b_cx2_1, W_cx2_2, b_cx2_2,
           W_cx3_1, b_cx3_1, W_cx3_2, b_cx3_2,
           W_lin1, b_lin1, W_lin2, b_lin2,
           W_lin3, b_lin3, W_lin4, b_lin4):
    B = state.shape[0]
    INP = state.shape[1] + task_indicator.shape[1]
    KPAD = ((INP + 127) // 128) * 128
    R = 256
    H2, H1, NH = 1024, 512, 64

    a = jnp.concatenate(
        [state, task_indicator,
         jnp.zeros((B, KPAD - INP), jnp.float32)], axis=1
    ).astype(jnp.bfloat16)

    w11 = _wk(W_cx1_1, KPAD)
    b11 = b_cx1_1[None, :]
    w21 = _wk(W_cx2_1, KPAD)
    b21 = b_cx2_1[None, :]
    w31 = jnp.pad(_wk(W_cx3_1, KPAD), ((0, 64), (0, 0)))
    b31 = jnp.pad(b_cx3_1, (0, 64))[None, :]
    wl1 = _wk(W_lin1, KPAD)
    bl1 = b_lin1[None, :]

    w12 = W_cx1_2.astype(jnp.bfloat16)
    b12 = b_cx1_2[None, :]
    w22 = W_cx2_2.astype(jnp.bfloat16)
    b22 = b_cx2_2[None, :]
    w32 = jnp.pad(W_cx3_2, ((0, 64), (0, 64))).astype(jnp.bfloat16)
    b32 = jnp.pad(b_cx3_2, (0, 64), constant_values=-1e9)[None, :]
    wl2 = W_lin2.astype(jnp.bfloat16)
    bl2 = b_lin2[None, :]
    wl3 = jnp.pad(W_lin3, ((0, 64), (0, 0))).astype(jnp.bfloat16)
    bl3 = jnp.pad(b_lin3, (0, 64))[None, :]
    wl4 = jnp.pad(W_lin4, ((0, 0), (0, 64))).astype(jnp.bfloat16)
    bl4 = b_lin4[None, :]

    t1, t2, t3 = _tri(H2), _tri(H1), _tri(128)

    def const(shape):
        return pl.BlockSpec(shape, lambda i: (0, 0))

    out = pl.pallas_call(
        _body,
        grid=(B // R,),
        in_specs=[
            pl.BlockSpec((R, KPAD), lambda i: (i, 0)),
            const(w11.shape), const(b11.shape), const(w12.shape), const(b12.shape),
            const(w21.shape), const(b21.shape), const(w22.shape), const(b22.shape),
            const(w31.shape), const(b31.shape), const(w32.shape), const(b32.shape),
            const(wl1.shape), const(bl1.shape), const(wl2.shape), const(bl2.shape),
            const(wl3.shape), const(bl3.shape), const(wl4.shape), const(bl4.shape),
            const(t1.shape), const(t2.shape), const(t3.shape),
        ],
        out_specs=pl.BlockSpec((R, NH), lambda i: (i, 0)),
        out_shape=jax.ShapeDtypeStruct((B, NH), jnp.float32),
    )(a, w11, b11, w12, b12, w21, b21, w22, b22, w31, b31, w32, b32,
      wl1, bl1, wl2, bl2, wl3, bl3, wl4, bl4, t1, t2, t3)
    return out


# speedup vs baseline: 1.3646x; 1.0849x over previous
import jax
import jax.numpy as jnp
import numpy as np
from jax.experimental import pallas as pl

_MININT = np.int32(-2147483648)
_MAXPOS = np.int32(2147483647)


def _dott(x, w):
    return jax.lax.dot_general(x, w, (((1,), (1,)), ((), ())),
                               preferred_element_type=jnp.float32)


def _kwta(x, key_src, k, tri_bf16):
    skey = jax.lax.bitcast_convert_type(key_src + 0.0, jnp.int32)
    skey = jnp.where(skey < 0, skey ^ _MAXPOS, skey)

    skey_t = skey.T
    k_row = k.T

    def body(i, t_u):
        bit = jax.lax.shift_left(jnp.int32(1), jnp.int32(31) - i)
        cand = t_u | bit
        thr = cand ^ _MININT
        cnt = jnp.sum((skey_t >= thr).astype(jnp.int32), axis=0, keepdims=True)
        return jnp.where(cnt >= k_row, cand, t_u)

    t_u = jax.lax.fori_loop(0, 32, body, jnp.zeros_like(k_row))
    t_s = (t_u ^ _MININT).T

    gt = skey > t_s
    c_gt = jnp.sum(gt.astype(jnp.int32), axis=1, keepdims=True)
    eq = skey == t_s
    cum_excl = jnp.dot(eq.astype(jnp.bfloat16), tri_bf16,
                       preferred_element_type=jnp.float32)
    keep = eq & (cum_excl < (k - c_gt).astype(jnp.float32))
    mask = (gt | keep) & (k > 0)
    return jnp.where(mask, x, x / 3.0)


def _body(a_ref, at_ref, w11_ref, w11t_ref, b11_ref, w12_ref, b12_ref,
          w21_ref, w21t_ref, b21_ref, w22_ref, b22_ref,
          w31_ref, w31t_ref, b31_ref, w32_ref, b32_ref,
          wl1_ref, wl1t_ref, bl1_ref, wl2_ref, bl2_ref,
          wl3_ref, bl3_ref, wl4_ref, bl4_ref,
          t1_ref, t2_ref, t3_ref, out_ref):
    f32 = jnp.float32
    a = a_ref[...]
    at = at_ref[...]

    def in_dot(wm_ref, wt_ref):
        return _dott(a, wm_ref[...]) + _dott(at, wt_ref[...])

    h1 = jnp.tanh(in_dot(w11_ref, w11t_ref) + b11_ref[...])
    z1 = _dott(h1.astype(jnp.bfloat16), w12_ref[...]) + b12_ref[...]
    k1 = jnp.argmax(z1, axis=1).astype(jnp.int32)[:, None]

    h2 = jnp.tanh(in_dot(w21_ref, w21t_ref) + b21_ref[...])
    z2 = _dott(h2.astype(jnp.bfloat16), w22_ref[...]) + b22_ref[...]
    k2 = jnp.argmax(z2, axis=1).astype(jnp.int32)[:, None]

    h3 = jnp.tanh(in_dot(w31_ref, w31t_ref) + b31_ref[...])
    z3 = _dott(h3.astype(jnp.bfloat16), w32_ref[...]) + b32_ref[...]
    k3 = jnp.argmax(z3, axis=1).astype(jnp.int32)[:, None]

    x = in_dot(wl1_ref, wl1t_ref) + bl1_ref[...]
    x = _kwta(x, x, k1, t1_ref[...])
    x = _dott(x.astype(jnp.bfloat16), wl2_ref[...]) + bl2_ref[...]
    x = _kwta(x, x, k2, t2_ref[...])
    x = _dott(x.astype(jnp.bfloat16), wl3_ref[...]) + bl3_ref[...]
    col = jax.lax.broadcasted_iota(jnp.int32, x.shape, 1)
    key3 = jnp.where(col < 64, x, f32(-1e30))
    x = _kwta(x, key3, k3, t3_ref[...])
    out_ref[...] = _dott(x.astype(jnp.bfloat16), wl4_ref[...]) + bl4_ref[...]


def _tri(n):
    r = jnp.arange(n, dtype=jnp.int32)
    return (r[:, None] < r[None, :]).astype(jnp.bfloat16)


def _wsplit(W, kmain):
    wm = W[:, :kmain].astype(jnp.bfloat16)
    wt = jnp.pad(W[:, kmain:], ((0, 0), (0, 128 - (W.shape[1] - kmain)))
                 ).astype(jnp.bfloat16)
    return wm, wt


def kernel(state, task_indicator,
           W_cx1_1, b_cx1_1, W_cx1_2, b_cx1_2,
           W_cx2_1, b_cx2_1, W_cx2_2, b_cx2_2,
           W_cx3_1, b_cx3_1, W_cx3_2, b_cx3_2,
           W_lin1, b_lin1, W_lin2, b_lin2,
           W_lin3, b_lin3, W_lin4, b_lin4):
    B = state.shape[0]
    NS = state.shape[1]
    KM = 2 * NS
    R = 256
    H2, H1, NH = 1024, 512, 64

    a = jnp.concatenate([state, task_indicator[:, :NS]],
                        axis=1).astype(jnp.bfloat16)
    at = jnp.pad(task_indicator[:, NS:],
                 ((0, 0), (0, 128 - (task_indicator.shape[1] - NS)))
                 ).astype(jnp.bfloat16)

    w11, w11t = _wsplit(W_cx1_1, KM)
    b11 = b_cx1_1[None, :]
    w21, w21t = _wsplit(W_cx2_1, KM)
    b21 = b_cx2_1[None, :]
    w31, w31t = _wsplit(jnp.pad(W_cx3_1, ((0, 64), (0, 0))), KM)
    b31 = jnp.pad(b_cx3_1, (0, 64))[None, :]
    wl1, wl1t = _wsplit(W_lin1, KM)
    bl1 = b_lin1[None, :]

    w12 = W_cx1_2.astype(jnp.bfloat16)
    b12 = b_cx1_2[None, :]
    w22 = W_cx2_2.astype(jnp.bfloat16)
    b22 = b_cx2_2[None, :]
    w32 = jnp.pad(W_cx3_2, ((0, 64), (0, 64))).astype(jnp.bfloat16)
    b32 = jnp.pad(b_cx3_2, (0, 64), constant_values=-1e9)[None, :]
    wl2 = W_lin2.astype(jnp.bfloat16)
    bl2 = b_lin2[None, :]
    wl3 = jnp.pad(W_lin3, ((0, 64), (0, 0))).astype(jnp.bfloat16)
    bl3 = jnp.pad(b_lin3, (0, 64))[None, :]
    wl4 = jnp.pad(W_lin4, ((0, 0), (0, 64))).astype(jnp.bfloat16)
    bl4 = b_lin4[None, :]

    t1, t2, t3 = _tri(H2), _tri(H1), _tri(128)

    def const(shape):
        return pl.BlockSpec(shape, lambda i: (0, 0))

    out = pl.pallas_call(
        _body,
        grid=(B // R,),
        in_specs=[
            pl.BlockSpec((R, KM), lambda i: (i, 0)),
            pl.BlockSpec((R, 128), lambda i: (i, 0)),
            const(w11.shape), const(w11t.shape), const(b11.shape),
            const(w12.shape), const(b12.shape),
            const(w21.shape), const(w21t.shape), const(b21.shape),
            const(w22.shape), const(b22.shape),
            const(w31.shape), const(w31t.shape), const(b31.shape),
            const(w32.shape), const(b32.shape),
            const(wl1.shape), const(wl1t.shape), const(bl1.shape),
            const(wl2.shape), const(bl2.shape),
            const(wl3.shape), const(bl3.shape), const(wl4.shape), const(bl4.shape),
            const(t1.shape), const(t2.shape), const(t3.shape),
        ],
        out_specs=pl.BlockSpec((R, NH), lambda i: (i, 0)),
        out_shape=jax.ShapeDtypeStruct((B, NH), jnp.float32),
    )(a, at, w11, w11t, b11, w12, b12, w21, w21t, b21, w22, b22,
      w31, w31t, b31, w32, b32, wl1, wl1t, bl1, wl2, bl2,
      wl3, bl3, wl4, bl4, t1, t2, t3)
    return out


# speedup vs baseline: 1.4243x; 1.0438x over previous
import jax
import jax.numpy as jnp
import numpy as np
from jax.experimental import pallas as pl

_MININT = np.int32(-2147483648)
_MAXPOS = np.int32(2147483647)


def _dott(x, w):
    return jax.lax.dot_general(x, w, (((1,), (1,)), ((), ())),
                               preferred_element_type=jnp.float32)


def _kwta(x, key_src, k, tri_bf16):
    skey = jax.lax.bitcast_convert_type(key_src + 0.0, jnp.int32)
    skey = jnp.where(skey < 0, skey ^ _MAXPOS, skey)

    skey_t = skey.T
    k_row = k.T

    def body(i, t_u):
        bit = jax.lax.shift_left(jnp.int32(1), jnp.int32(31) - i)
        cand = t_u | bit
        thr = cand ^ _MININT
        cnt = jnp.sum((skey_t >= thr).astype(jnp.int32), axis=0, keepdims=True)
        return jnp.where(cnt >= k_row, cand, t_u)

    t_u = jax.lax.fori_loop(0, 32, body, jnp.zeros_like(k_row))
    t_s = (t_u ^ _MININT).T

    gt = skey > t_s
    c_gt = jnp.sum(gt.astype(jnp.int32), axis=1, keepdims=True)
    eq = skey == t_s
    cum_excl = jnp.dot(eq.astype(jnp.bfloat16), tri_bf16,
                       preferred_element_type=jnp.float32)
    keep = eq & (cum_excl < (k - c_gt).astype(jnp.float32))
    mask = (gt | keep) & (k > 0)
    return jnp.where(mask, x, x / 3.0)


def _body(s_ref, ti_ref, at_ref,
          w11a_ref, w11b_ref, w11t_ref, b11_ref, w12_ref, b12_ref,
          w21a_ref, w21b_ref, w21t_ref, b21_ref, w22_ref, b22_ref,
          w31a_ref, w31b_ref, w31t_ref, b31_ref, w32_ref, b32_ref,
          wl1a_ref, wl1b_ref, wl1t_ref, bl1_ref, wl2_ref, bl2_ref,
          wl3_ref, bl3_ref, wl4_ref, bl4_ref,
          t1_ref, t2_ref, t3_ref, out_ref):
    f32 = jnp.float32
    sa = s_ref[...].astype(jnp.bfloat16)
    tb = ti_ref[...].astype(jnp.bfloat16)
    at = at_ref[...]

    def in_dot(wa_ref, wb_ref, wt_ref):
        return (_dott(sa, wa_ref[...]) + _dott(tb, wb_ref[...])
                + _dott(at, wt_ref[...]))

    h1 = jnp.tanh(in_dot(w11a_ref, w11b_ref, w11t_ref) + b11_ref[...])
    z1 = _dott(h1.astype(jnp.bfloat16), w12_ref[...]) + b12_ref[...]
    k1 = jnp.argmax(z1, axis=1).astype(jnp.int32)[:, None]

    h2 = jnp.tanh(in_dot(w21a_ref, w21b_ref, w21t_ref) + b21_ref[...])
    z2 = _dott(h2.astype(jnp.bfloat16), w22_ref[...]) + b22_ref[...]
    k2 = jnp.argmax(z2, axis=1).astype(jnp.int32)[:, None]

    h3 = jnp.tanh(in_dot(w31a_ref, w31b_ref, w31t_ref) + b31_ref[...])
    z3 = _dott(h3.astype(jnp.bfloat16), w32_ref[...]) + b32_ref[...]
    k3 = jnp.argmax(z3, axis=1).astype(jnp.int32)[:, None]

    x = in_dot(wl1a_ref, wl1b_ref, wl1t_ref) + bl1_ref[...]
    x = _kwta(x, x, k1, t1_ref[...])
    x = _dott(x.astype(jnp.bfloat16), wl2_ref[...]) + bl2_ref[...]
    x = _kwta(x, x, k2, t2_ref[...])
    x = _dott(x.astype(jnp.bfloat16), wl3_ref[...]) + bl3_ref[...]
    col = jax.lax.broadcasted_iota(jnp.int32, x.shape, 1)
    key3 = jnp.where(col < 64, x, f32(-1e30))
    x = _kwta(x, key3, k3, t3_ref[...])
    out_ref[...] = _dott(x.astype(jnp.bfloat16), wl4_ref[...]) + bl4_ref[...]


def _tri(n):
    r = jnp.arange(n, dtype=jnp.int32)
    return (r[:, None] < r[None, :]).astype(jnp.bfloat16)


def _wsplit(W, ns):
    wa = W[:, :ns].astype(jnp.bfloat16)
    wb = W[:, ns:2 * ns].astype(jnp.bfloat16)
    wt = jnp.pad(W[:, 2 * ns:], ((0, 0), (0, 128 - (W.shape[1] - 2 * ns)))
                 ).astype(jnp.bfloat16)
    return wa, wb, wt


def kernel(state, task_indicator,
           W_cx1_1, b_cx1_1, W_cx1_2, b_cx1_2,
           W_cx2_1, b_cx2_1, W_cx2_2, b_cx2_2,
           W_cx3_1, b_cx3_1, W_cx3_2, b_cx3_2,
           W_lin1, b_lin1, W_lin2, b_lin2,
           W_lin3, b_lin3, W_lin4, b_lin4):
    B = state.shape[0]
    NS = state.shape[1]
    KM = 2 * NS
    R = 256
    H2, H1, NH = 1024, 512, 64

    at = jnp.pad(task_indicator[:, NS:],
                 ((0, 0), (0, 128 - (task_indicator.shape[1] - NS)))
                 ).astype(jnp.bfloat16)

    w11a, w11b, w11t = _wsplit(W_cx1_1, NS)
    b11 = b_cx1_1[None, :]
    w21a, w21b, w21t = _wsplit(W_cx2_1, NS)
    b21 = b_cx2_1[None, :]
    w31a, w31b, w31t = _wsplit(jnp.pad(W_cx3_1, ((0, 64), (0, 0))), NS)
    b31 = jnp.pad(b_cx3_1, (0, 64))[None, :]
    wl1a, wl1b, wl1t = _wsplit(W_lin1, NS)
    bl1 = b_lin1[None, :]

    w12 = W_cx1_2.astype(jnp.bfloat16)
    b12 = b_cx1_2[None, :]
    w22 = W_cx2_2.astype(jnp.bfloat16)
    b22 = b_cx2_2[None, :]
    w32 = jnp.pad(W_cx3_2, ((0, 64), (0, 64))).astype(jnp.bfloat16)
    b32 = jnp.pad(b_cx3_2, (0, 64), constant_values=-1e9)[None, :]
    wl2 = W_lin2.astype(jnp.bfloat16)
    bl2 = b_lin2[None, :]
    wl3 = jnp.pad(W_lin3, ((0, 64), (0, 0))).astype(jnp.bfloat16)
    bl3 = jnp.pad(b_lin3, (0, 64))[None, :]
    wl4 = jnp.pad(W_lin4, ((0, 0), (0, 64))).astype(jnp.bfloat16)
    bl4 = b_lin4[None, :]

    t1, t2, t3 = _tri(H2), _tri(H1), _tri(128)

    def const(shape):
        return pl.BlockSpec(shape, lambda i: (0, 0))

    out = pl.pallas_call(
        _body,
        grid=(B // R,),
        in_specs=[
            pl.BlockSpec((R, NS), lambda i: (i, 0)),
            pl.BlockSpec((R, NS), lambda i: (i, 0)),
            pl.BlockSpec((R, 128), lambda i: (i, 0)),
            const(w11a.shape), const(w11b.shape), const(w11t.shape),
            const(b11.shape), const(w12.shape), const(b12.shape),
            const(w21a.shape), const(w21b.shape), const(w21t.shape),
            const(b21.shape), const(w22.shape), const(b22.shape),
            const(w31a.shape), const(w31b.shape), const(w31t.shape),
            const(b31.shape), const(w32.shape), const(b32.shape),
            const(wl1a.shape), const(wl1b.shape), const(wl1t.shape),
            const(bl1.shape), const(wl2.shape), const(bl2.shape),
            const(wl3.shape), const(bl3.shape), const(wl4.shape), const(bl4.shape),
            const(t1.shape), const(t2.shape), const(t3.shape),
        ],
        out_specs=pl.BlockSpec((R, NH), lambda i: (i, 0)),
        out_shape=jax.ShapeDtypeStruct((B, NH), jnp.float32),
    )(state, task_indicator, at,
      w11a, w11b, w11t, b11, w12, b12,
      w21a, w21b, w21t, b21, w22, b22,
      w31a, w31b, w31t, b31, w32, b32,
      wl1a, wl1b, wl1t, bl1, wl2, bl2,
      wl3, bl3, wl4, bl4, t1, t2, t3)
    return out


# speedup vs baseline: 1.6252x; 1.1411x over previous
import jax
import jax.numpy as jnp
import numpy as np
from jax.experimental import pallas as pl

_MININT = np.int32(-2147483648)
_MAXPOS = np.int32(2147483647)


def _dott(x, w):
    return jax.lax.dot_general(x, w, (((1,), (1,)), ((), ())),
                               preferred_element_type=jnp.float32)


def _kwta(x, key_src, k, tri_bf16):
    skey = jax.lax.bitcast_convert_type(key_src + 0.0, jnp.int32)
    skey = jnp.where(skey < 0, skey ^ _MAXPOS, skey)

    skey_t = skey.T
    k_row = k.T

    def body(i, t_u):
        bit = jax.lax.shift_left(jnp.int32(1), jnp.int32(31) - i)
        cand = t_u | bit
        thr = cand ^ _MININT
        cnt = jnp.sum((skey_t >= thr).astype(jnp.int32), axis=0, keepdims=True)
        return jnp.where(cnt >= k_row, cand, t_u)

    t_u = jax.lax.fori_loop(0, 32, body, jnp.zeros_like(k_row))
    t_s = (t_u ^ _MININT).T

    gt = skey > t_s
    c_gt = jnp.sum(gt.astype(jnp.int32), axis=1, keepdims=True)
    eq = skey == t_s
    cum_excl = jnp.dot(eq.astype(jnp.bfloat16), tri_bf16,
                       preferred_element_type=jnp.float32)
    keep = eq & (cum_excl < (k - c_gt).astype(jnp.float32))
    mask = (gt | keep) & (k > 0)
    return jnp.where(mask, x, x / 3.0)


def _body(s_ref, ti_ref, at_ref,
          w11a_ref, w11b_ref, w11t_ref, b11_ref, w12_ref, b12_ref,
          w21a_ref, w21b_ref, w21t_ref, b21_ref, w22_ref, b22_ref,
          w31a_ref, w31b_ref, w31t_ref, b31_ref, w32_ref, b32_ref,
          wl1a_ref, wl1b_ref, wl1t_ref, bl1_ref, wl2_ref, bl2_ref,
          wl3_ref, bl3_ref, wl4_ref, bl4_ref,
          t1_ref, t2_ref, t3_ref, out_ref):
    f32 = jnp.float32
    sa = s_ref[...].astype(jnp.bfloat16)
    tb = ti_ref[...].astype(jnp.bfloat16)
    at = at_ref[...]

    def in_dot(wa_ref, wb_ref, wt_ref):
        return (_dott(sa, wa_ref[...]) + _dott(tb, wb_ref[...])
                + _dott(at, wt_ref[...]))

    h1 = jnp.tanh(in_dot(w11a_ref, w11b_ref, w11t_ref) + b11_ref[...])
    z1 = _dott(h1.astype(jnp.bfloat16), w12_ref[...]) + b12_ref[...]
    k1 = jnp.argmax(z1, axis=1).astype(jnp.int32)[:, None]

    h2 = jnp.tanh(in_dot(w21a_ref, w21b_ref, w21t_ref) + b21_ref[...])
    z2 = _dott(h2.astype(jnp.bfloat16), w22_ref[...]) + b22_ref[...]
    k2 = jnp.argmax(z2, axis=1).astype(jnp.int32)[:, None]

    h3 = jnp.tanh(in_dot(w31a_ref, w31b_ref, w31t_ref) + b31_ref[...])
    z3 = _dott(h3.astype(jnp.bfloat16), w32_ref[...]) + b32_ref[...]
    k3 = jnp.argmax(z3, axis=1).astype(jnp.int32)[:, None]

    x = in_dot(wl1a_ref, wl1b_ref, wl1t_ref) + bl1_ref[...]
    x = _kwta(x, x, k1, t1_ref[...])
    x = _dott(x.astype(jnp.bfloat16), wl2_ref[...]) + bl2_ref[...]
    x = _kwta(x, x, k2, t2_ref[...])
    x = _dott(x.astype(jnp.bfloat16), wl3_ref[...]) + bl3_ref[...]
    col = jax.lax.broadcasted_iota(jnp.int32, x.shape, 1)
    key3 = jnp.where(col < 64, x, f32(-1e30))
    x = _kwta(x, key3, k3, t3_ref[...])
    out_ref[...] = _dott(x.astype(jnp.bfloat16), wl4_ref[...]) + bl4_ref[...]


def _tri(n):
    r = jnp.arange(n, dtype=jnp.int32)
    return (r[:, None] < r[None, :]).astype(jnp.bfloat16)


def _wsplit(W, ns):
    wa = W[:, :ns].astype(jnp.bfloat16)
    wb = W[:, ns:2 * ns].astype(jnp.bfloat16)
    wt = jnp.pad(W[:, 2 * ns:], ((0, 0), (0, 128 - (W.shape[1] - 2 * ns)))
                 ).astype(jnp.bfloat16)
    return wa, wb, wt


def kernel(state, task_indicator,
           W_cx1_1, b_cx1_1, W_cx1_2, b_cx1_2,
           W_cx2_1, b_cx2_1, W_cx2_2, b_cx2_2,
           W_cx3_1, b_cx3_1, W_cx3_2, b_cx3_2,
           W_lin1, b_lin1, W_lin2, b_lin2,
           W_lin3, b_lin3, W_lin4, b_lin4):
    B = state.shape[0]
    NS = state.shape[1]
    KM = 2 * NS
    R = 512
    H2, H1, NH = 1024, 512, 64

    at = jnp.pad(task_indicator[:, NS:],
                 ((0, 0), (0, 128 - (task_indicator.shape[1] - NS)))
                 ).astype(jnp.bfloat16)

    w11a, w11b, w11t = _wsplit(W_cx1_1, NS)
    b11 = b_cx1_1[None, :]
    w21a, w21b, w21t = _wsplit(W_cx2_1, NS)
    b21 = b_cx2_1[None, :]
    w31a, w31b, w31t = _wsplit(jnp.pad(W_cx3_1, ((0, 64), (0, 0))), NS)
    b31 = jnp.pad(b_cx3_1, (0, 64))[None, :]
    wl1a, wl1b, wl1t = _wsplit(W_lin1, NS)
    bl1 = b_lin1[None, :]

    w12 = W_cx1_2.astype(jnp.bfloat16)
    b12 = b_cx1_2[None, :]
    w22 = W_cx2_2.astype(jnp.bfloat16)
    b22 = b_cx2_2[None, :]
    w32 = jnp.pad(W_cx3_2, ((0, 64), (0, 64))).astype(jnp.bfloat16)
    b32 = jnp.pad(b_cx3_2, (0, 64), constant_values=-1e9)[None, :]
    wl2 = W_lin2.astype(jnp.bfloat16)
    bl2 = b_lin2[None, :]
    wl3 = jnp.pad(W_lin3, ((0, 64), (0, 0))).astype(jnp.bfloat16)
    bl3 = jnp.pad(b_lin3, (0, 64))[None, :]
    wl4 = jnp.pad(W_lin4, ((0, 0), (0, 64))).astype(jnp.bfloat16)
    bl4 = b_lin4[None, :]

    t1, t2, t3 = _tri(H2), _tri(H1), _tri(128)

    def const(shape):
        return pl.BlockSpec(shape, lambda i: (0, 0))

    out = pl.pallas_call(
        _body,
        grid=(B // R,),
        in_specs=[
            pl.BlockSpec((R, NS), lambda i: (i, 0)),
            pl.BlockSpec((R, NS), lambda i: (i, 0)),
            pl.BlockSpec((R, 128), lambda i: (i, 0)),
            const(w11a.shape), const(w11b.shape), const(w11t.shape),
            const(b11.shape), const(w12.shape), const(b12.shape),
            const(w21a.shape), const(w21b.shape), const(w21t.shape),
            const(b21.shape), const(w22.shape), const(b22.shape),
            const(w31a.shape), const(w31b.shape), const(w31t.shape),
            const(b31.shape), const(w32.shape), const(b32.shape),
            const(wl1a.shape), const(wl1b.shape), const(wl1t.shape),
            const(bl1.shape), const(wl2.shape), const(bl2.shape),
            const(wl3.shape), const(bl3.shape), const(wl4.shape), const(bl4.shape),
            const(t1.shape), const(t2.shape), const(t3.shape),
        ],
        out_specs=pl.BlockSpec((R, NH), lambda i: (i, 0)),
        out_shape=jax.ShapeDtypeStruct((B, NH), jnp.float32),
    )(state, task_indicator, at,
      w11a, w11b, w11t, b11, w12, b12,
      w21a, w21b, w21t, b21, w22, b22,
      w31a, w31b, w31t, b31, w32, b32,
      wl1a, wl1b, wl1t, bl1, wl2, bl2,
      wl3, bl3, wl4, bl4, t1, t2, t3)
    return out


# speedup vs baseline: 1.6380x; 1.0079x over previous
import jax
import jax.numpy as jnp
import numpy as np
from jax.experimental import pallas as pl

_MININT = np.int32(-2147483648)
_MAXPOS = np.int32(2147483647)


def _dott(x, w):
    return jax.lax.dot_general(x, w, (((1,), (1,)), ((), ())),
                               preferred_element_type=jnp.float32)


def _kwta(x, key_src, k, tri_bf16):
    skey = jax.lax.bitcast_convert_type(key_src + 0.0, jnp.int32)
    skey = jnp.where(skey < 0, skey ^ _MAXPOS, skey)

    skey_t = skey.T
    k_row = k.T

    def body(i, t_u):
        bit = jax.lax.shift_left(jnp.int32(1), jnp.int32(31) - i)
        cand = t_u | bit
        thr = cand ^ _MININT
        cnt = jnp.sum((skey_t >= thr).astype(jnp.int32), axis=0, keepdims=True)
        return jnp.where(cnt >= k_row, cand, t_u)

    t_u = jax.lax.fori_loop(0, 32, body, jnp.zeros_like(k_row), unroll=4)
    t_s = (t_u ^ _MININT).T

    gt = skey > t_s
    c_gt = jnp.sum(gt.astype(jnp.int32), axis=1, keepdims=True)
    eq = skey == t_s
    cum_excl = jnp.dot(eq.astype(jnp.bfloat16), tri_bf16,
                       preferred_element_type=jnp.float32)
    keep = eq & (cum_excl < (k - c_gt).astype(jnp.float32))
    mask = (gt | keep) & (k > 0)
    return jnp.where(mask, x, x / 3.0)


def _body(s_ref, ti_ref, at_ref,
          w11a_ref, w11b_ref, w11t_ref, b11_ref, w12_ref, b12_ref,
          w21a_ref, w21b_ref, w21t_ref, b21_ref, w22_ref, b22_ref,
          w31a_ref, w31b_ref, w31t_ref, b31_ref, w32_ref, b32_ref,
          wl1a_ref, wl1b_ref, wl1t_ref, bl1_ref, wl2_ref, bl2_ref,
          wl3_ref, bl3_ref, wl4_ref, bl4_ref,
          t1_ref, t2_ref, t3_ref, out_ref):
    f32 = jnp.float32
    sa = s_ref[...].astype(jnp.bfloat16)
    tb = ti_ref[...].astype(jnp.bfloat16)
    at = at_ref[...]

    def in_dot(wa_ref, wb_ref, wt_ref):
        return (_dott(sa, wa_ref[...]) + _dott(tb, wb_ref[...])
                + _dott(at, wt_ref[...]))

    h1 = jnp.tanh(in_dot(w11a_ref, w11b_ref, w11t_ref) + b11_ref[...])
    z1 = _dott(h1.astype(jnp.bfloat16), w12_ref[...]) + b12_ref[...]
    k1 = jnp.argmax(z1, axis=1).astype(jnp.int32)[:, None]

    h2 = jnp.tanh(in_dot(w21a_ref, w21b_ref, w21t_ref) + b21_ref[...])
    z2 = _dott(h2.astype(jnp.bfloat16), w22_ref[...]) + b22_ref[...]
    k2 = jnp.argmax(z2, axis=1).astype(jnp.int32)[:, None]

    h3 = jnp.tanh(in_dot(w31a_ref, w31b_ref, w31t_ref) + b31_ref[...])
    z3 = _dott(h3.astype(jnp.bfloat16), w32_ref[...]) + b32_ref[...]
    k3 = jnp.argmax(z3, axis=1).astype(jnp.int32)[:, None]

    x = in_dot(wl1a_ref, wl1b_ref, wl1t_ref) + bl1_ref[...]
    x = _kwta(x, x, k1, t1_ref[...])
    x = _dott(x.astype(jnp.bfloat16), wl2_ref[...]) + bl2_ref[...]
    x = _kwta(x, x, k2, t2_ref[...])
    x = _dott(x.astype(jnp.bfloat16), wl3_ref[...]) + bl3_ref[...]
    col = jax.lax.broadcasted_iota(jnp.int32, x.shape, 1)
    key3 = jnp.where(col < 64, x, f32(-1e30))
    x = _kwta(x, key3, k3, t3_ref[...])
    out_ref[...] = _dott(x.astype(jnp.bfloat16), wl4_ref[...]) + bl4_ref[...]


def _tri(n):
    r = jnp.arange(n, dtype=jnp.int32)
    return (r[:, None] < r[None, :]).astype(jnp.bfloat16)


def _wsplit(W, ns):
    wa = W[:, :ns].astype(jnp.bfloat16)
    wb = W[:, ns:2 * ns].astype(jnp.bfloat16)
    wt = jnp.pad(W[:, 2 * ns:], ((0, 0), (0, 128 - (W.shape[1] - 2 * ns)))
                 ).astype(jnp.bfloat16)
    return wa, wb, wt


def kernel(state, task_indicator,
           W_cx1_1, b_cx1_1, W_cx1_2, b_cx1_2,
           W_cx2_1, b_cx2_1, W_cx2_2, b_cx2_2,
           W_cx3_1, b_cx3_1, W_cx3_2, b_cx3_2,
           W_lin1, b_lin1, W_lin2, b_lin2,
           W_lin3, b_lin3, W_lin4, b_lin4):
    B = state.shape[0]
    NS = state.shape[1]
    KM = 2 * NS
    R = 512
    H2, H1, NH = 1024, 512, 64

    at = jnp.pad(task_indicator[:, NS:],
                 ((0, 0), (0, 128 - (task_indicator.shape[1] - NS)))
                 ).astype(jnp.bfloat16)

    w11a, w11b, w11t = _wsplit(W_cx1_1, NS)
    b11 = b_cx1_1[None, :]
    w21a, w21b, w21t = _wsplit(W_cx2_1, NS)
    b21 = b_cx2_1[None, :]
    w31a, w31b, w31t = _wsplit(jnp.pad(W_cx3_1, ((0, 64), (0, 0))), NS)
    b31 = jnp.pad(b_cx3_1, (0, 64))[None, :]
    wl1a, wl1b, wl1t = _wsplit(W_lin1, NS)
    bl1 = b_lin1[None, :]

    w12 = W_cx1_2.astype(jnp.bfloat16)
    b12 = b_cx1_2[None, :]
    w22 = W_cx2_2.astype(jnp.bfloat16)
    b22 = b_cx2_2[None, :]
    w32 = jnp.pad(W_cx3_2, ((0, 64), (0, 64))).astype(jnp.bfloat16)
    b32 = jnp.pad(b_cx3_2, (0, 64), constant_values=-1e9)[None, :]
    wl2 = W_lin2.astype(jnp.bfloat16)
    bl2 = b_lin2[None, :]
    wl3 = jnp.pad(W_lin3, ((0, 64), (0, 0))).astype(jnp.bfloat16)
    bl3 = jnp.pad(b_lin3, (0, 64))[None, :]
    wl4 = jnp.pad(W_lin4, ((0, 0), (0, 64))).astype(jnp.bfloat16)
    bl4 = b_lin4[None, :]

    t1, t2, t3 = _tri(H2), _tri(H1), _tri(128)

    def const(shape):
        return pl.BlockSpec(shape, lambda i: (0, 0))

    out = pl.pallas_call(
        _body,
        grid=(B // R,),
        in_specs=[
            pl.BlockSpec((R, NS), lambda i: (i, 0)),
            pl.BlockSpec((R, NS), lambda i: (i, 0)),
            pl.BlockSpec((R, 128), lambda i: (i, 0)),
            const(w11a.shape), const(w11b.shape), const(w11t.shape),
            const(b11.shape), const(w12.shape), const(b12.shape),
            const(w21a.shape), const(w21b.shape), const(w21t.shape),
            const(b21.shape), const(w22.shape), const(b22.shape),
            const(w31a.shape), const(w31b.shape), const(w31t.shape),
            const(b31.shape), const(w32.shape), const(b32.shape),
            const(wl1a.shape), const(wl1b.shape), const(wl1t.shape),
            const(bl1.shape), const(wl2.shape), const(bl2.shape),
            const(wl3.shape), const(bl3.shape), const(wl4.shape), const(bl4.shape),
            const(t1.shape), const(t2.shape), const(t3.shape),
        ],
        out_specs=pl.BlockSpec((R, NH), lambda i: (i, 0)),
        out_shape=jax.ShapeDtypeStruct((B, NH), jnp.float32),
    )(state, task_indicator, at,
      w11a, w11b, w11t, b11, w12, b12,
      w21a, w21b, w21t, b21, w22, b22,
      w31a, w31b, w31t, b31, w32, b32,
      wl1a, wl1b, wl1t, bl1, wl2, bl2,
      wl3, bl3, wl4, bl4, t1, t2, t3)
    return out
